# Initial kernel scaffold; baseline (speedup 1.0000x reference)
import jax
import jax.numpy as jnp
from jax.experimental import pallas as pl


def kernel(x, W, node2graph, edge_list1, edge_list2):
    raise NotImplementedError("write your pallas kernel here")


# speedup vs baseline: 2.1198x; 2.1198x over previous
import functools

import jax
import jax.numpy as jnp
from jax import lax
from jax.experimental import pallas as pl
from jax.experimental.pallas import tpu as pltpu
from jax.experimental.pallas import tpu_sc as plsc

N_NODES = 50000
NGRAPH = 128
DIM = 256
E_TOT = 1600000

NTILE = 32
CHUNK = E_TOT // NTILE
LCHUNK = CHUNK // 16
JB = 256
NBLK_FULL = LCHUNK // JB
TAIL = LCHUNK - NBLK_FULL * JB
D_AL = [(5 * l) % 8 for l in range(16)]
LB = ((3 * (7 + JB) + 7) // 8) * 8
TAIL_SZ = [((3 * (D_AL[l] + TAIL) + 7) // 8) * 8 for l in range(16)]
SSTEP = 64

_SC_PARAMS = pltpu.CompilerParams(needs_layout_passes=False,
                                  use_tc_tiling_on_sc=False)


@functools.cache
def _mesh():
    return plsc.VectorSubcoreMesh(core_axis_name="c", subcore_axis_name="s",
                                  num_cores=2, num_subcores=16)


def _lane_vecs():
    iota = lax.iota(jnp.int32, 16)
    dvec = (iota * 5) & 7
    loff = iota * LB + dvec * 3
    return iota, loff


def _fire_block(el1, el2, is1, base_w, j0, tail, buf, sem):
    def fire(el):
        for l in range(16):
            st = base_w + (l * LCHUNK - D_AL[l]) * 3 + j0 * 3
            st = pl.multiple_of(st, 8)
            sz = TAIL_SZ[l] if tail else LB
            pltpu.async_copy(el.at[pl.ds(st, sz)],
                             buf.at[pl.ds(l * LB, sz)], sem)

    @pl.when(is1)
    def _():
        fire(el1)

    @pl.when(jnp.logical_not(is1))
    def _():
        fire(el2)


def _wait_block(el1, buf, sem, tail):
    for l in range(16):
        sz = TAIL_SZ[l] if tail else LB
        pltpu.make_async_copy(el1.at[pl.ds(0, sz)],
                              buf.at[pl.ds(l * LB, sz)], sem).wait()


def _hist_body(el1, el2, n2g_hbm, hist_hbm, n2g_v, buf0, buf1, ctr,
               semn, sem0, sem1):
    c = lax.axis_index("c")
    s = lax.axis_index("s")
    t = c * 16 + s
    is1 = t < 16
    tloc = jnp.where(is1, t, t - 16)
    base_w = tloc * (CHUNK * 3)

    cpn = pltpu.async_copy(n2g_hbm, n2g_v, semn)
    zz = jnp.zeros((16,), jnp.int32)
    for l in range(16):
        for bg in range(8):
            ctr[l, pl.ds(bg * 16, 16)] = zz

    iota, loff = _lane_vecs()
    ones_i = jnp.full((16,), 1, jnp.int32)
    bufs = [buf0, buf1]
    sems = [sem0, sem1]
    _fire_block(el1, el2, is1, base_w, 0, False, buf0, sem0)
    _fire_block(el1, el2, is1, base_w, JB, False, buf1, sem1)
    cpn.wait()

    def hstep(j, carry, buf):
        idx0 = loff + j * 3
        src = plsc.load_gather(buf, [idx0])
        g = plsc.load_gather(n2g_v, [src])
        plsc.addupdate_scatter(ctr, [iota, g], ones_i)
        return carry

    def blk_body(i, carry):
        for b in range(2):
            kk = 2 * i + b
            _wait_block(el1, bufs[b], sems[b], False)
            lax.fori_loop(0, JB, functools.partial(hstep, buf=bufs[b]), 0)

            @pl.when(kk + 2 < NBLK_FULL)
            def _(kk=kk, b=b):
                _fire_block(el1, el2, is1, base_w, (kk + 2) * JB, False,
                            bufs[b], sems[b])

            @pl.when(kk == NBLK_FULL - 2)
            def _(b=b):
                _fire_block(el1, el2, is1, base_w, NBLK_FULL * JB, True,
                            bufs[b], sems[b])
        return carry

    lax.fori_loop(0, NBLK_FULL // 2, blk_body, 0)
    _wait_block(el1, buf0, sem0, True)
    lax.fori_loop(0, TAIL, functools.partial(hstep, buf=buf0), 0)
    pltpu.sync_copy(ctr, hist_hbm.at[t])


@functools.cache
def _hist_call():
    return pl.kernel(
        _hist_body,
        out_type=jax.ShapeDtypeStruct((NTILE, 16, NGRAPH), jnp.int32),
        mesh=_mesh(),
        compiler_params=_SC_PARAMS,
        scratch_types=[
            pltpu.VMEM((N_NODES,), jnp.int32),
            pltpu.VMEM((16 * LB,), jnp.int32),
            pltpu.VMEM((16 * LB,), jnp.int32),
            pltpu.VMEM((16, NGRAPH), jnp.int32),
            pltpu.SemaphoreType.DMA,
            pltpu.SemaphoreType.DMA,
            pltpu.SemaphoreType.DMA,
        ],
    )


def _place_body(el1, el2, n2g_hbm, hist_hbm,
                out_el, out_off, out_ne,
                n2g_v, buf0, buf1, ctr, starts_v, histp0, histp1, ne_stage,
                ix_s0, ix_s1, ix_d0, ix_d1, ix_r0, ix_r1, ix_o0, ix_o1,
                d_src, d_dst, d_rel, d_off,
                semn, sem0, sem1, semh0, semh1, semsc0, semsc1):
    c = lax.axis_index("c")
    s = lax.axis_index("s")
    t = c * 16 + s
    is1 = t < 16
    tloc = jnp.where(is1, t, t - 16)
    base_w = tloc * (CHUNK * 3)
    iota, loff = _lane_vecs()
    zeros16 = jnp.zeros((16,), jnp.int32)

    cpn = pltpu.async_copy(n2g_hbm, n2g_v, semn)

    histp = [histp0, histp1]
    semh = [semh0, semh1]
    pltpu.async_copy(hist_hbm.at[0], histp0, semh0)
    pltpu.async_copy(hist_hbm.at[1], histp1, semh1)

    def scan_body(i, acc):
        for b in range(2):
            tp = 2 * i + b
            pltpu.make_async_copy(hist_hbm.at[0], histp[b], semh[b]).wait()
            for l in range(16):
                @pl.when(tp == t)
                def _(l=l, acc=acc):
                    for bg in range(8):
                        ctr[l, pl.ds(bg * 16, 16)] = acc[bg]
                row = [histp[b][l, pl.ds(bg * 16, 16)] for bg in range(8)]
                acc = tuple(acc[bg] + row[bg] for bg in range(8))

            @pl.when(tp + 2 < NTILE)
            def _(b=b, tp=tp):
                pltpu.async_copy(hist_hbm.at[tp + 2], histp[b], semh[b])
        return acc

    acc = lax.fori_loop(0, NTILE // 2, scan_body,
                        tuple(zeros16 for _ in range(8)))

    for bg in range(8):
        ne_stage[pl.ds(bg * 16, 16)] = acc[bg]

    @pl.when(t == 0)
    def _():
        pltpu.sync_copy(ne_stage, out_ne)

    carry = jnp.int32(0)
    for bg in range(8):
        inc = plsc.cumsum(acc[bg])
        excl = inc - acc[bg] + carry
        carry = carry + jnp.sum(acc[bg])
        for l in range(16):
            ctr[l, pl.ds(bg * 16, 16)] = ctr[l, pl.ds(bg * 16, 16)] + excl

    cpn.wait()
    for bg in range(8):
        bvals = iota + bg * 16

        def bs_body(_, lohi, bvals=bvals):
            lo, hi = lohi
            mid = (lo + hi) >> 1
            v = plsc.load_gather(n2g_v, [mid])
            pred = v < bvals
            return (jnp.where(pred, mid + 1, lo), jnp.where(pred, hi, mid))

        lo, hi = lax.fori_loop(0, 17, bs_body,
                               (zeros16, jnp.full((16,), N_NODES, jnp.int32)))
        starts_v[pl.ds(bg * 16, 16)] = lo

    roff = jnp.where(is1, jnp.int32(0), jnp.int32(4))
    bufs = [buf0, buf1]
    sems = [sem0, sem1]
    semsc = [semsc0, semsc1]
    ix_s = [ix_s0, ix_s1]
    ix_d = [ix_d0, ix_d1]
    ix_r = [ix_r0, ix_r1]
    ix_o = [ix_o0, ix_o1]

    def step2(j2, carry, buf, jbase, p):
        jj = jbase + j2
        idx0 = loff + jj * 3
        src = plsc.load_gather(buf, [idx0])
        dst = plsc.load_gather(buf, [idx0 + 1])
        rel = plsc.load_gather(buf, [idx0 + 2]) + roff
        g = plsc.load_gather(n2g_v, [src])
        pos = plsc.load_gather(ctr, [iota, g])
        plsc.store_scatter(ctr, [iota, g], pos + 1)
        soff = plsc.load_gather(starts_v, [g])
        col = j2 * 16 + iota
        p3 = pos * 3
        plsc.store_scatter(ix_s[p], [col], p3)
        plsc.store_scatter(ix_d[p], [col], p3 + 1)
        plsc.store_scatter(ix_r[p], [col], p3 + 2)
        plsc.store_scatter(ix_o[p], [col], pos)
        dcol = p * 128 + col
        plsc.store_scatter(d_src, [dcol], src)
        plsc.store_scatter(d_dst, [dcol], dst)
        plsc.store_scatter(d_rel, [dcol], rel)
        plsc.store_scatter(d_off, [dcol], soff)
        return carry

    def fire_stage(p):
        sl = pl.ds(p * 128, 128)
        return [pltpu.async_copy(d_src.at[sl], out_el.at[ix_s[p]], semsc[p]),
                pltpu.async_copy(d_dst.at[sl], out_el.at[ix_d[p]], semsc[p]),
                pltpu.async_copy(d_rel.at[sl], out_el.at[ix_r[p]], semsc[p]),
                pltpu.async_copy(d_off.at[sl], out_off.at[ix_o[p]], semsc[p])]

    def wait_stage(p):
        sl = pl.ds(p * 128, 128)
        pltpu.make_async_copy(d_src.at[sl], out_el.at[ix_s[p]],
                              semsc[p]).wait()
        pltpu.make_async_copy(d_dst.at[sl], out_el.at[ix_d[p]],
                              semsc[p]).wait()
        pltpu.make_async_copy(d_rel.at[sl], out_el.at[ix_r[p]],
                              semsc[p]).wait()
        pltpu.make_async_copy(d_off.at[sl], out_off.at[ix_o[p]],
                              semsc[p]).wait()

    _fire_block(el1, el2, is1, base_w, 0, False, buf0, sem0)
    _fire_block(el1, el2, is1, base_w, JB, False, buf1, sem1)

    NSTG = JB // 8 // 2

    def blk_body(i, carry):
        for b in range(2):
            kk = 2 * i + b
            _wait_block(el1, bufs[b], sems[b], False)

            def stage_pair(sp, carry2, buf=bufs[b], kk=kk):
                for p in range(2):
                    gidx = kk * (2 * NSTG) + sp * 2 + p

                    @pl.when(gidx >= 2)
                    def _(p=p):
                        wait_stage(p)

                    lax.fori_loop(0, 8,
                                  functools.partial(step2, buf=buf,
                                                    jbase=(sp * 2 + p) * 8,
                                                    p=p),
                                  0)
                    fire_stage(p)
                return carry2

            lax.fori_loop(0, NSTG, stage_pair, 0)

            @pl.when(kk + 2 < NBLK_FULL)
            def _(kk=kk, b=b):
                _fire_block(el1, el2, is1, base_w, (kk + 2) * JB, False,
                            bufs[b], sems[b])

            @pl.when(kk == NBLK_FULL - 2)
            def _(b=b):
                _fire_block(el1, el2, is1, base_w, NBLK_FULL * JB, True,
                            bufs[b], sems[b])
        return carry

    lax.fori_loop(0, NBLK_FULL // 2, blk_body, 0)

    _wait_block(el1, buf0, sem0, True)

    def tail_pair(sp, carry2):
        for p in range(2):
            wait_stage(p)
            lax.fori_loop(0, 8,
                          functools.partial(step2, buf=buf0,
                                            jbase=(sp * 2 + p) * 8, p=p),
                          0)
            fire_stage(p)
        return carry2

    lax.fori_loop(0, 3, tail_pair, 0)
    wait_stage(0)
    lax.fori_loop(0, 5,
                  functools.partial(step2, buf=buf0, jbase=48, p=0), 0)
    tail_cps = []
    for w in range(5):
        sl = pl.ds(w * 16, 16)
        tail_cps.append(pltpu.async_copy(
            d_src.at[sl], out_el.at[ix_s0[sl]], semsc0))
        tail_cps.append(pltpu.async_copy(
            d_dst.at[sl], out_el.at[ix_d0[sl]], semsc0))
        tail_cps.append(pltpu.async_copy(
            d_rel.at[sl], out_el.at[ix_r0[sl]], semsc0))
        tail_cps.append(pltpu.async_copy(
            d_off.at[sl], out_off.at[ix_o0[sl]], semsc0))
    wait_stage(1)
    for cp in tail_cps:
        cp.wait()


@functools.cache
def _place_call():
    return pl.kernel(
        _place_body,
        out_type=(jax.ShapeDtypeStruct((E_TOT * 3,), jnp.int32),
                  jax.ShapeDtypeStruct((E_TOT,), jnp.int32),
                  jax.ShapeDtypeStruct((NGRAPH,), jnp.int32)),
        mesh=_mesh(),
        compiler_params=_SC_PARAMS,
        scratch_types=[
            pltpu.VMEM((N_NODES,), jnp.int32),
            pltpu.VMEM((16 * LB,), jnp.int32),
            pltpu.VMEM((16 * LB,), jnp.int32),
            pltpu.VMEM((16, NGRAPH), jnp.int32),
            pltpu.VMEM((NGRAPH,), jnp.int32),
            pltpu.VMEM((16, NGRAPH), jnp.int32),
            pltpu.VMEM((16, NGRAPH), jnp.int32),
            pltpu.VMEM((NGRAPH,), jnp.int32),
            pltpu.VMEM((128,), jnp.int32),
            pltpu.VMEM((128,), jnp.int32),
            pltpu.VMEM((128,), jnp.int32),
            pltpu.VMEM((128,), jnp.int32),
            pltpu.VMEM((128,), jnp.int32),
            pltpu.VMEM((128,), jnp.int32),
            pltpu.VMEM((128,), jnp.int32),
            pltpu.VMEM((128,), jnp.int32),
            pltpu.VMEM((256,), jnp.int32),
            pltpu.VMEM((256,), jnp.int32),
            pltpu.VMEM((256,), jnp.int32),
            pltpu.VMEM((256,), jnp.int32),
            pltpu.SemaphoreType.DMA,
            pltpu.SemaphoreType.DMA,
            pltpu.SemaphoreType.DMA,
            pltpu.SemaphoreType.DMA,
            pltpu.SemaphoreType.DMA,
            pltpu.SemaphoreType.DMA,
            pltpu.SemaphoreType.DMA,
        ],
    )


def _mm_body(x_ref, w_ref, h_ref, ew_ref):
    h_ref[...] = jnp.maximum(
        jnp.dot(x_ref[...], w_ref[...], preferred_element_type=jnp.float32),
        0.0)
    ew_ref[...] = jnp.ones_like(ew_ref)


_mm_call = pl.pallas_call(
    _mm_body,
    grid=(50,),
    in_specs=[pl.BlockSpec((1000, DIM), lambda i: (i, 0)),
              pl.BlockSpec((DIM, DIM), lambda i: (0, 0))],
    out_specs=[pl.BlockSpec((1000, DIM), lambda i: (i, 0)),
               pl.BlockSpec((8, 4000), lambda i: (i, 0))],
    out_shape=[jax.ShapeDtypeStruct((N_NODES, DIM), jnp.float32),
               jax.ShapeDtypeStruct((400, 4000), jnp.float32)],
)


def kernel(x, W, node2graph, edge_list1, edge_list2):
    n2g = node2graph.astype(jnp.int32)
    el1 = edge_list1.astype(jnp.int32).reshape(-1)
    el2 = edge_list2.astype(jnp.int32).reshape(-1)
    hist = _hist_call()(el1, el2, n2g)
    elflat, offsets, num_edges = _place_call()(el1, el2, n2g, hist)
    h, ew = _mm_call(x, W)
    edge_weight = ew.reshape(-1)
    out_el = elflat.reshape(E_TOT, 3)
    num_relation = jnp.array(8, jnp.int32)
    return (h, out_el, edge_weight, num_edges, offsets, num_relation)


# speedup vs baseline: 2.1207x; 1.0004x over previous
import functools

import jax
import jax.numpy as jnp
from jax import lax
from jax.experimental import pallas as pl
from jax.experimental.pallas import tpu as pltpu
from jax.experimental.pallas import tpu_sc as plsc

N_NODES = 50000
NGRAPH = 128
DIM = 256
E_TOT = 1600000

NTILE = 32
CHUNK = E_TOT // NTILE
LCHUNK = CHUNK // 16
JB = 256
NBLK_FULL = LCHUNK // JB
TAIL = LCHUNK - NBLK_FULL * JB
D_AL = [(5 * l) % 8 for l in range(16)]
LB = ((3 * (7 + JB) + 7) // 8) * 8
TAIL_SZ = [((3 * (D_AL[l] + TAIL) + 7) // 8) * 8 for l in range(16)]
SSTEP = 64

_SC_PARAMS = pltpu.CompilerParams(needs_layout_passes=False,
                                  use_tc_tiling_on_sc=False)


@functools.cache
def _mesh():
    return plsc.VectorSubcoreMesh(core_axis_name="c", subcore_axis_name="s",
                                  num_cores=2, num_subcores=16)


def _lane_vecs():
    iota = lax.iota(jnp.int32, 16)
    dvec = (iota * 5) & 7
    loff = iota * LB + dvec * 3
    return iota, loff


def _fire_block(el1, el2, is1, base_w, j0, tail, buf, sem):
    def fire(el):
        for l in range(16):
            st = base_w + (l * LCHUNK - D_AL[l]) * 3 + j0 * 3
            st = pl.multiple_of(st, 8)
            sz = TAIL_SZ[l] if tail else LB
            pltpu.async_copy(el.at[pl.ds(st, sz)],
                             buf.at[pl.ds(l * LB, sz)], sem)

    @pl.when(is1)
    def _():
        fire(el1)

    @pl.when(jnp.logical_not(is1))
    def _():
        fire(el2)


def _wait_block(el1, buf, sem, tail):
    for l in range(16):
        sz = TAIL_SZ[l] if tail else LB
        pltpu.make_async_copy(el1.at[pl.ds(0, sz)],
                              buf.at[pl.ds(l * LB, sz)], sem).wait()


def _hist_body(el1, el2, n2g_hbm, hist_hbm, n2g_v, buf0, buf1, ctr,
               semn, sem0, sem1):
    c = lax.axis_index("c")
    s = lax.axis_index("s")
    t = c * 16 + s
    is1 = t < 16
    tloc = jnp.where(is1, t, t - 16)
    base_w = tloc * (CHUNK * 3)

    cpn = pltpu.async_copy(n2g_hbm, n2g_v, semn)
    zz = jnp.zeros((16,), jnp.int32)
    for l in range(16):
        for bg in range(8):
            ctr[l, pl.ds(bg * 16, 16)] = zz

    iota, loff = _lane_vecs()
    ones_i = jnp.full((16,), 1, jnp.int32)
    bufs = [buf0, buf1]
    sems = [sem0, sem1]
    _fire_block(el1, el2, is1, base_w, 0, False, buf0, sem0)
    _fire_block(el1, el2, is1, base_w, JB, False, buf1, sem1)
    cpn.wait()

    def hstep(j, carry, buf):
        idx0 = loff + j * 3
        src = plsc.load_gather(buf, [idx0])
        g = plsc.load_gather(n2g_v, [src])
        plsc.addupdate_scatter(ctr, [iota, g], ones_i)
        return carry

    def blk_body(i, carry):
        for b in range(2):
            kk = 2 * i + b
            _wait_block(el1, bufs[b], sems[b], False)
            lax.fori_loop(0, JB, functools.partial(hstep, buf=bufs[b]), 0)

            @pl.when(kk + 2 < NBLK_FULL)
            def _(kk=kk, b=b):
                _fire_block(el1, el2, is1, base_w, (kk + 2) * JB, False,
                            bufs[b], sems[b])

            @pl.when(kk == NBLK_FULL - 2)
            def _(b=b):
                _fire_block(el1, el2, is1, base_w, NBLK_FULL * JB, True,
                            bufs[b], sems[b])
        return carry

    lax.fori_loop(0, NBLK_FULL // 2, blk_body, 0)
    _wait_block(el1, buf0, sem0, True)
    lax.fori_loop(0, TAIL, functools.partial(hstep, buf=buf0), 0)
    pltpu.sync_copy(ctr, hist_hbm.at[t])


@functools.cache
def _hist_call():
    return pl.kernel(
        _hist_body,
        out_type=jax.ShapeDtypeStruct((NTILE, 16, NGRAPH), jnp.int32),
        mesh=_mesh(),
        compiler_params=_SC_PARAMS,
        scratch_types=[
            pltpu.VMEM((N_NODES,), jnp.int32),
            pltpu.VMEM((16 * LB,), jnp.int32),
            pltpu.VMEM((16 * LB,), jnp.int32),
            pltpu.VMEM((16, NGRAPH), jnp.int32),
            pltpu.SemaphoreType.DMA,
            pltpu.SemaphoreType.DMA,
            pltpu.SemaphoreType.DMA,
        ],
    )


NPAR = 8


def _place_body(el1, el2, n2g_hbm, hist_hbm,
                out_el, out_off, out_ne, *rest):
    it = iter(rest)
    n2g_v, buf0, buf1, ctr, starts_v, histp0, histp1, ne_stage = (
        next(it) for _ in range(8))
    ix = [[next(it) for _ in range(4)] for _ in range(NPAR)]
    d_src, d_dst, d_rel, d_off = (next(it) for _ in range(4))
    semn, sem0, sem1, semh0, semh1 = (next(it) for _ in range(5))
    semsc = [next(it) for _ in range(NPAR)]
    c = lax.axis_index("c")
    s = lax.axis_index("s")
    t = c * 16 + s
    is1 = t < 16
    tloc = jnp.where(is1, t, t - 16)
    base_w = tloc * (CHUNK * 3)
    iota, loff = _lane_vecs()
    zeros16 = jnp.zeros((16,), jnp.int32)

    cpn = pltpu.async_copy(n2g_hbm, n2g_v, semn)

    histp = [histp0, histp1]
    semh = [semh0, semh1]
    pltpu.async_copy(hist_hbm.at[0], histp0, semh0)
    pltpu.async_copy(hist_hbm.at[1], histp1, semh1)

    def scan_body(i, acc):
        for b in range(2):
            tp = 2 * i + b
            pltpu.make_async_copy(hist_hbm.at[0], histp[b], semh[b]).wait()
            for l in range(16):
                @pl.when(tp == t)
                def _(l=l, acc=acc):
                    for bg in range(8):
                        ctr[l, pl.ds(bg * 16, 16)] = acc[bg]
                row = [histp[b][l, pl.ds(bg * 16, 16)] for bg in range(8)]
                acc = tuple(acc[bg] + row[bg] for bg in range(8))

            @pl.when(tp + 2 < NTILE)
            def _(b=b, tp=tp):
                pltpu.async_copy(hist_hbm.at[tp + 2], histp[b], semh[b])
        return acc

    acc = lax.fori_loop(0, NTILE // 2, scan_body,
                        tuple(zeros16 for _ in range(8)))

    for bg in range(8):
        ne_stage[pl.ds(bg * 16, 16)] = acc[bg]

    @pl.when(t == 0)
    def _():
        pltpu.sync_copy(ne_stage, out_ne)

    carry = jnp.int32(0)
    for bg in range(8):
        inc = plsc.cumsum(acc[bg])
        excl = inc - acc[bg] + carry
        carry = carry + jnp.sum(acc[bg])
        for l in range(16):
            ctr[l, pl.ds(bg * 16, 16)] = ctr[l, pl.ds(bg * 16, 16)] + excl

    cpn.wait()
    for bg in range(8):
        bvals = iota + bg * 16

        def bs_body(_, lohi, bvals=bvals):
            lo, hi = lohi
            mid = (lo + hi) >> 1
            v = plsc.load_gather(n2g_v, [mid])
            pred = v < bvals
            return (jnp.where(pred, mid + 1, lo), jnp.where(pred, hi, mid))

        lo, hi = lax.fori_loop(0, 17, bs_body,
                               (zeros16, jnp.full((16,), N_NODES, jnp.int32)))
        starts_v[pl.ds(bg * 16, 16)] = lo

    roff = jnp.where(is1, jnp.int32(0), jnp.int32(4))
    bufs = [buf0, buf1]
    sems = [sem0, sem1]

    def step2(j2, carry, buf, jbase, p):
        jj = jbase + j2
        idx0 = loff + jj * 3
        src = plsc.load_gather(buf, [idx0])
        dst = plsc.load_gather(buf, [idx0 + 1])
        rel = plsc.load_gather(buf, [idx0 + 2]) + roff
        g = plsc.load_gather(n2g_v, [src])
        pos = plsc.load_gather(ctr, [iota, g])
        plsc.store_scatter(ctr, [iota, g], pos + 1)
        soff = plsc.load_gather(starts_v, [g])
        col = j2 * 16 + iota
        p3 = pos * 3
        plsc.store_scatter(ix[p][0], [col], p3)
        plsc.store_scatter(ix[p][1], [col], p3 + 1)
        plsc.store_scatter(ix[p][2], [col], p3 + 2)
        plsc.store_scatter(ix[p][3], [col], pos)
        dcol = p * 128 + col
        plsc.store_scatter(d_src, [dcol], src)
        plsc.store_scatter(d_dst, [dcol], dst)
        plsc.store_scatter(d_rel, [dcol], rel)
        plsc.store_scatter(d_off, [dcol], soff)
        return carry

    def fire_stage(p):
        sl = pl.ds(p * 128, 128)
        return [pltpu.async_copy(d_src.at[sl], out_el.at[ix[p][0]], semsc[p]),
                pltpu.async_copy(d_dst.at[sl], out_el.at[ix[p][1]], semsc[p]),
                pltpu.async_copy(d_rel.at[sl], out_el.at[ix[p][2]], semsc[p]),
                pltpu.async_copy(d_off.at[sl], out_off.at[ix[p][3]],
                                 semsc[p])]

    def wait_stage(p):
        sl = pl.ds(p * 128, 128)
        pltpu.make_async_copy(d_src.at[sl], out_el.at[ix[p][0]],
                              semsc[p]).wait()
        pltpu.make_async_copy(d_dst.at[sl], out_el.at[ix[p][1]],
                              semsc[p]).wait()
        pltpu.make_async_copy(d_rel.at[sl], out_el.at[ix[p][2]],
                              semsc[p]).wait()
        pltpu.make_async_copy(d_off.at[sl], out_off.at[ix[p][3]],
                              semsc[p]).wait()

    _fire_block(el1, el2, is1, base_w, 0, False, buf0, sem0)
    _fire_block(el1, el2, is1, base_w, JB, False, buf1, sem1)

    NSG = JB // 8 // NPAR

    def blk_body(i, carry):
        for b in range(2):
            kk = 2 * i + b
            _wait_block(el1, bufs[b], sems[b], False)

            def stage_grp(sg, carry2, buf=bufs[b], kk=kk):
                for p in range(NPAR):
                    gidx = kk * (JB // 8) + sg * NPAR + p

                    @pl.when(gidx >= NPAR)
                    def _(p=p):
                        wait_stage(p)

                    lax.fori_loop(0, 8,
                                  functools.partial(step2, buf=buf,
                                                    jbase=(sg * NPAR + p) * 8,
                                                    p=p),
                                  0)
                    fire_stage(p)
                return carry2

            lax.fori_loop(0, NSG, stage_grp, 0)

            @pl.when(kk + 2 < NBLK_FULL)
            def _(kk=kk, b=b):
                _fire_block(el1, el2, is1, base_w, (kk + 2) * JB, False,
                            bufs[b], sems[b])

            @pl.when(kk == NBLK_FULL - 2)
            def _(b=b):
                _fire_block(el1, el2, is1, base_w, NBLK_FULL * JB, True,
                            bufs[b], sems[b])
        return carry

    lax.fori_loop(0, NBLK_FULL // 2, blk_body, 0)

    _wait_block(el1, buf0, sem0, True)
    for q in range(6):
        wait_stage(q)
        lax.fori_loop(0, 8,
                      functools.partial(step2, buf=buf0, jbase=q * 8, p=q), 0)
        fire_stage(q)
    wait_stage(6)
    lax.fori_loop(0, 5,
                  functools.partial(step2, buf=buf0, jbase=48, p=6), 0)
    tail_cps = []
    for w in range(5):
        sl6 = pl.ds(6 * 128 + w * 16, 16)
        sl = pl.ds(w * 16, 16)
        tail_cps.append(pltpu.async_copy(
            d_src.at[sl6], out_el.at[ix[6][0][sl]], semsc[6]))
        tail_cps.append(pltpu.async_copy(
            d_dst.at[sl6], out_el.at[ix[6][1][sl]], semsc[6]))
        tail_cps.append(pltpu.async_copy(
            d_rel.at[sl6], out_el.at[ix[6][2][sl]], semsc[6]))
        tail_cps.append(pltpu.async_copy(
            d_off.at[sl6], out_off.at[ix[6][3][sl]], semsc[6]))
    wait_stage(7)
    for q in range(6):
        wait_stage(q)
    for cp in tail_cps:
        cp.wait()


@functools.cache
def _place_call():
    return pl.kernel(
        _place_body,
        out_type=(jax.ShapeDtypeStruct((E_TOT * 3,), jnp.int32),
                  jax.ShapeDtypeStruct((E_TOT,), jnp.int32),
                  jax.ShapeDtypeStruct((NGRAPH,), jnp.int32)),
        mesh=_mesh(),
        compiler_params=_SC_PARAMS,
        scratch_types=[
            pltpu.VMEM((N_NODES,), jnp.int32),
            pltpu.VMEM((16 * LB,), jnp.int32),
            pltpu.VMEM((16 * LB,), jnp.int32),
            pltpu.VMEM((16, NGRAPH), jnp.int32),
            pltpu.VMEM((NGRAPH,), jnp.int32),
            pltpu.VMEM((16, NGRAPH), jnp.int32),
            pltpu.VMEM((16, NGRAPH), jnp.int32),
            pltpu.VMEM((NGRAPH,), jnp.int32),
        ] + [pltpu.VMEM((128,), jnp.int32) for _ in range(4 * NPAR)]
          + [pltpu.VMEM((NPAR * 128,), jnp.int32) for _ in range(4)]
          + [pltpu.SemaphoreType.DMA for _ in range(5 + NPAR)],
    )


def _mm_body(x_ref, w_ref, h_ref):
    h_ref[...] = jnp.maximum(
        jnp.dot(x_ref[...], w_ref[...], preferred_element_type=jnp.float32),
        0.0)


_mm_call = pl.pallas_call(
    _mm_body,
    grid=(50,),
    in_specs=[pl.BlockSpec((1000, DIM), lambda i: (i, 0)),
              pl.BlockSpec((DIM, DIM), lambda i: (0, 0))],
    out_specs=[pl.BlockSpec((1000, DIM), lambda i: (i, 0))],
    out_shape=[jax.ShapeDtypeStruct((N_NODES, DIM), jnp.float32)],
)


def _ones_body(ew_ref):
    ew_ref[...] = jnp.ones_like(ew_ref)


_ones_call = pl.pallas_call(
    _ones_body,
    grid=(1,),
    out_specs=[pl.BlockSpec((E_TOT // 128, 128), lambda i: (0, 0))],
    out_shape=[jax.ShapeDtypeStruct((E_TOT // 128, 128), jnp.float32)],
)


def kernel(x, W, node2graph, edge_list1, edge_list2):
    n2g = node2graph.astype(jnp.int32)
    el1 = edge_list1.astype(jnp.int32).reshape(-1)
    el2 = edge_list2.astype(jnp.int32).reshape(-1)
    hist = _hist_call()(el1, el2, n2g)
    elflat, offsets, num_edges = _place_call()(el1, el2, n2g, hist)
    (h,) = _mm_call(x, W)
    (ew2,) = _ones_call()
    edge_weight = ew2.reshape(-1)
    out_el = elflat.reshape(E_TOT, 3)
    num_relation = jnp.array(8, jnp.int32)
    return (h, out_el, edge_weight, num_edges, offsets, num_relation)


# speedup vs baseline: 2.3315x; 1.0994x over previous
import functools

import jax
import jax.numpy as jnp
from jax import lax
from jax.experimental import pallas as pl
from jax.experimental.pallas import tpu as pltpu
from jax.experimental.pallas import tpu_sc as plsc

N_NODES = 50000
NGRAPH = 128
DIM = 256
E_TOT = 1600000

NTILE = 32
CHUNK = E_TOT // NTILE
LCHUNK = CHUNK // 16
JB = 128
NBLK_FULL = LCHUNK // JB
TAIL = LCHUNK - NBLK_FULL * JB
D_AL = [(5 * l) % 8 for l in range(16)]
NRB = ((7 + JB + 7) // 8) * 8
TAIL_R = [((D_AL[l] + TAIL + 7) // 8) * 8 for l in range(16)]

_SC_PARAMS = pltpu.CompilerParams(needs_layout_passes=False,
                                  use_tc_tiling_on_sc=False)


@functools.cache
def _mesh():
    return plsc.VectorSubcoreMesh(core_axis_name="c", subcore_axis_name="s",
                                  num_cores=2, num_subcores=16)


def _lane_vecs():
    iota = lax.iota(jnp.int32, 16)
    dvec = (iota * 5) & 7
    return iota, dvec


def _fire_block(el1, el2, is1, base_r, j0, tail, buf, sem):
    def fire(el):
        for l in range(16):
            r0 = base_r + (l * LCHUNK - D_AL[l]) + j0
            r0 = pl.multiple_of(r0, 8)
            nr = TAIL_R[l] if tail else NRB
            pltpu.async_copy(el.at[pl.ds(r0, nr)],
                             buf.at[l, pl.ds(0, nr)], sem)

    @pl.when(is1)
    def _():
        fire(el1)

    @pl.when(jnp.logical_not(is1))
    def _():
        fire(el2)


def _wait_block(el1, buf, sem, tail):
    for l in range(16):
        nr = TAIL_R[l] if tail else NRB
        pltpu.make_async_copy(el1.at[pl.ds(0, nr)],
                              buf.at[l, pl.ds(0, nr)], sem).wait()


def _hist_body(el1, el2, n2g_hbm, hist_hbm, n2g_v, buf0, buf1, ctr,
               semn, sem0, sem1):
    c = lax.axis_index("c")
    s = lax.axis_index("s")
    t = c * 16 + s
    is1 = t < 16
    tloc = jnp.where(is1, t, t - 16)
    base_r = tloc * CHUNK

    cpn = pltpu.async_copy(n2g_hbm, n2g_v, semn)
    zz = jnp.zeros((16,), jnp.int32)
    for l in range(16):
        for bg in range(8):
            ctr[l, pl.ds(bg * 16, 16)] = zz

    iota, dvec = _lane_vecs()
    ones_i = jnp.full((16,), 1, jnp.int32)
    zeros16 = jnp.zeros((16,), jnp.int32)
    bufs = [buf0, buf1]
    sems = [sem0, sem1]
    _fire_block(el1, el2, is1, base_r, 0, False, buf0, sem0)
    _fire_block(el1, el2, is1, base_r, JB, False, buf1, sem1)
    cpn.wait()

    def hstep(j, carry, buf):
        src = plsc.load_gather(buf, [iota, dvec + j, zeros16])
        g = plsc.load_gather(n2g_v, [src])
        plsc.addupdate_scatter(ctr, [iota, g], ones_i)
        return carry

    def blk_body(i, carry):
        for b in range(2):
            kk = 2 * i + b
            _wait_block(el1, bufs[b], sems[b], False)
            lax.fori_loop(0, JB, functools.partial(hstep, buf=bufs[b]), 0)

            @pl.when(kk + 2 < NBLK_FULL)
            def _(kk=kk, b=b):
                _fire_block(el1, el2, is1, base_r, (kk + 2) * JB, False,
                            bufs[b], sems[b])

            @pl.when(kk == NBLK_FULL - 2)
            def _(b=b):
                _fire_block(el1, el2, is1, base_r, NBLK_FULL * JB, True,
                            bufs[b], sems[b])
        return carry

    lax.fori_loop(0, NBLK_FULL // 2, blk_body, 0)
    _wait_block(el1, buf0, sem0, True)
    lax.fori_loop(0, TAIL, functools.partial(hstep, buf=buf0), 0)
    pltpu.sync_copy(ctr, hist_hbm.at[t])


@functools.cache
def _hist_call():
    return pl.kernel(
        _hist_body,
        out_type=jax.ShapeDtypeStruct((NTILE, 16, NGRAPH), jnp.int32),
        mesh=_mesh(),
        compiler_params=_SC_PARAMS,
        scratch_types=[
            pltpu.VMEM((N_NODES,), jnp.int32),
            pltpu.VMEM((16, NRB, 3), jnp.int32),
            pltpu.VMEM((16, NRB, 3), jnp.int32),
            pltpu.VMEM((16, NGRAPH), jnp.int32),
            pltpu.SemaphoreType.DMA,
            pltpu.SemaphoreType.DMA,
            pltpu.SemaphoreType.DMA,
        ],
    )


NPAR = 8


def _place_body(el1, el2, n2g_hbm, hist_hbm,
                out_el, out_off, out_ne, *rest):
    it = iter(rest)
    n2g_v, buf0, buf1, ctr, starts_v, histp0, histp1, ne_stage, gb_v = (
        next(it) for _ in range(9))
    vfill = [next(it), next(it)]
    ix = [[next(it) for _ in range(3)] for _ in range(NPAR)]
    d_src, d_dst, d_rel = (next(it) for _ in range(3))
    semn, sem0, sem1, semh0, semh1 = (next(it) for _ in range(5))
    semsc = [next(it) for _ in range(NPAR)]
    c = lax.axis_index("c")
    s = lax.axis_index("s")
    t = c * 16 + s
    is1 = t < 16
    tloc = jnp.where(is1, t, t - 16)
    base_r = tloc * CHUNK
    iota, dvec = _lane_vecs()
    zeros16 = jnp.zeros((16,), jnp.int32)
    ones16 = jnp.full((16,), 1, jnp.int32)
    twos16 = jnp.full((16,), 2, jnp.int32)

    cpn = pltpu.async_copy(n2g_hbm, n2g_v, semn)

    histp = [histp0, histp1]
    semh = [semh0, semh1]
    pltpu.async_copy(hist_hbm.at[0], histp0, semh0)
    pltpu.async_copy(hist_hbm.at[1], histp1, semh1)

    def scan_body(i, acc):
        for b in range(2):
            tp = 2 * i + b
            pltpu.make_async_copy(hist_hbm.at[0], histp[b], semh[b]).wait()
            for l in range(16):
                @pl.when(tp == t)
                def _(l=l, acc=acc):
                    for bg in range(8):
                        ctr[l, pl.ds(bg * 16, 16)] = acc[bg]
                row = [histp[b][l, pl.ds(bg * 16, 16)] for bg in range(8)]
                acc = tuple(acc[bg] + row[bg] for bg in range(8))

            @pl.when(tp + 2 < NTILE)
            def _(b=b, tp=tp):
                pltpu.async_copy(hist_hbm.at[tp + 2], histp[b], semh[b])
        return acc

    acc = lax.fori_loop(0, NTILE // 2, scan_body,
                        tuple(zeros16 for _ in range(8)))

    for bg in range(8):
        ne_stage[pl.ds(bg * 16, 16)] = acc[bg]

    @pl.when(t == 0)
    def _():
        pltpu.sync_copy(ne_stage, out_ne)

    carry = jnp.int32(0)
    for bg in range(8):
        inc = plsc.cumsum(acc[bg])
        excl = inc - acc[bg] + carry
        carry = carry + jnp.sum(acc[bg])
        gb_v[pl.ds(bg * 16, 16)] = excl
        for l in range(16):
            ctr[l, pl.ds(bg * 16, 16)] = ctr[l, pl.ds(bg * 16, 16)] + excl

    cpn.wait()
    for bg in range(8):
        bvals = iota + bg * 16

        def bs_body(_, lohi, bvals=bvals):
            lo, hi = lohi
            mid = (lo + hi) >> 1
            v = plsc.load_gather(n2g_v, [mid])
            pred = v < bvals
            return (jnp.where(pred, mid + 1, lo), jnp.where(pred, hi, mid))

        lo, hi = lax.fori_loop(0, 17, bs_body,
                               (zeros16, jnp.full((16,), N_NODES, jnp.int32)))
        starts_v[pl.ds(bg * 16, 16)] = lo

    def vscal(ref, b):
        return jnp.max(plsc.load_gather(ref, [jnp.broadcast_to(b, (16,))]))

    for k in range(4):
        bn = tloc * 4 + k + jnp.where(is1, 0, 64)
        vb = vfill[k & 1]
        sem = semsc[k & 1]
        lo = vscal(gb_v, bn)
        n = vscal(ne_stage, bn)
        vv = plsc.load_gather(starts_v, [jnp.broadcast_to(bn, (16,))])

        def fillbuf(i, carry, vb=vb, vv=vv):
            plsc.store_scatter(vb, [i * 16 + iota], vv)
            return carry

        lax.fori_loop(0, 64, fillbuf, 0)

        @pl.when(n > 0)
        def _(bn=bn, vb=vb, sem=sem, lo=lo, n=n):
            hi = lo + n
            idxf = jnp.minimum(lo + iota, hi - 1)
            cpf = pltpu.async_copy(vb.at[pl.ds(0, 16)], out_off.at[idxf], sem)
            idxb = jnp.maximum(hi - 16 + iota, lo)
            cpb = pltpu.async_copy(vb.at[pl.ds(0, 16)], out_off.at[idxb], sem)
            lo_a = (lo + 15) & ~15
            hi_a = hi & ~15
            nch = jnp.maximum(hi_a - lo_a, 0)
            nbig = nch >> 10
            n256 = (nch & 1023) >> 8
            n16 = (nch & 255) >> 4

            def fbig(q, carry, vb=vb, sem=sem, lo_a=lo_a):
                d0 = pl.multiple_of(lo_a + q * 1024, 16)
                pltpu.async_copy(vb, out_off.at[pl.ds(d0, 1024)], sem)
                return carry

            def f256(q, carry, vb=vb, sem=sem, base=lo_a + (nbig << 10)):
                d0 = pl.multiple_of(base + q * 256, 16)
                pltpu.async_copy(vb.at[pl.ds(0, 256)],
                                 out_off.at[pl.ds(d0, 256)], sem)
                return carry

            def f16(q, carry, vb=vb, sem=sem,
                    base=lo_a + (nbig << 10) + (n256 << 8)):
                d0 = pl.multiple_of(base + q * 16, 16)
                pltpu.async_copy(vb.at[pl.ds(0, 16)],
                                 out_off.at[pl.ds(d0, 16)], sem)
                return carry

            lax.fori_loop(0, nbig, fbig, 0)
            lax.fori_loop(0, n256, f256, 0)
            lax.fori_loop(0, n16, f16, 0)

            def wbig(q, carry, vb=vb, sem=sem):
                pltpu.make_async_copy(vb, out_off.at[pl.ds(0, 1024)],
                                      sem).wait()
                return carry

            def w256(q, carry, vb=vb, sem=sem):
                pltpu.make_async_copy(vb.at[pl.ds(0, 256)],
                                      out_off.at[pl.ds(0, 256)], sem).wait()
                return carry

            def w16(q, carry, vb=vb, sem=sem):
                pltpu.make_async_copy(vb.at[pl.ds(0, 16)],
                                      out_off.at[pl.ds(0, 16)], sem).wait()
                return carry

            lax.fori_loop(0, nbig, wbig, 0)
            lax.fori_loop(0, n256, w256, 0)
            lax.fori_loop(0, n16, w16, 0)
            cpf.wait()
            cpb.wait()

    roff = jnp.where(is1, jnp.int32(0), jnp.int32(4))
    bufs = [buf0, buf1]
    sems = [sem0, sem1]

    def step2(j2, carry, buf, jbase, p):
        jj = jbase + j2
        rowv = dvec + jj
        src = plsc.load_gather(buf, [iota, rowv, zeros16])
        dst = plsc.load_gather(buf, [iota, rowv, ones16])
        rel = plsc.load_gather(buf, [iota, rowv, twos16]) + roff
        g = plsc.load_gather(n2g_v, [src])
        pos = plsc.load_gather(ctr, [iota, g])
        plsc.store_scatter(ctr, [iota, g], pos + 1)
        col = j2 * 16 + iota
        p3 = pos * 3
        plsc.store_scatter(ix[p][0], [col], p3)
        plsc.store_scatter(ix[p][1], [col], p3 + 1)
        plsc.store_scatter(ix[p][2], [col], p3 + 2)
        dcol = p * 128 + col
        plsc.store_scatter(d_src, [dcol], src)
        plsc.store_scatter(d_dst, [dcol], dst)
        plsc.store_scatter(d_rel, [dcol], rel)
        return carry

    def fire_stage(p):
        sl = pl.ds(p * 128, 128)
        return [pltpu.async_copy(d_src.at[sl], out_el.at[ix[p][0]], semsc[p]),
                pltpu.async_copy(d_dst.at[sl], out_el.at[ix[p][1]], semsc[p]),
                pltpu.async_copy(d_rel.at[sl], out_el.at[ix[p][2]], semsc[p])]

    def wait_stage(p):
        sl = pl.ds(p * 128, 128)
        pltpu.make_async_copy(d_src.at[sl], out_el.at[ix[p][0]],
                              semsc[p]).wait()
        pltpu.make_async_copy(d_dst.at[sl], out_el.at[ix[p][1]],
                              semsc[p]).wait()
        pltpu.make_async_copy(d_rel.at[sl], out_el.at[ix[p][2]],
                              semsc[p]).wait()

    _fire_block(el1, el2, is1, base_r, 0, False, buf0, sem0)
    _fire_block(el1, el2, is1, base_r, JB, False, buf1, sem1)

    NSG = JB // 8 // NPAR

    def blk_body(i, carry):
        for b in range(2):
            kk = 2 * i + b
            _wait_block(el1, bufs[b], sems[b], False)

            def stage_grp(sg, carry2, buf=bufs[b], kk=kk):
                for p in range(NPAR):
                    gidx = kk * (JB // 8) + sg * NPAR + p

                    @pl.when(gidx >= NPAR)
                    def _(p=p):
                        wait_stage(p)

                    lax.fori_loop(0, 8,
                                  functools.partial(step2, buf=buf,
                                                    jbase=(sg * NPAR + p) * 8,
                                                    p=p),
                                  0)
                    fire_stage(p)
                return carry2

            lax.fori_loop(0, NSG, stage_grp, 0)

            @pl.when(kk + 2 < NBLK_FULL)
            def _(kk=kk, b=b):
                _fire_block(el1, el2, is1, base_r, (kk + 2) * JB, False,
                            bufs[b], sems[b])

            @pl.when(kk == NBLK_FULL - 2)
            def _(b=b):
                _fire_block(el1, el2, is1, base_r, NBLK_FULL * JB, True,
                            bufs[b], sems[b])
        return carry

    lax.fori_loop(0, NBLK_FULL // 2, blk_body, 0)

    _wait_block(el1, buf0, sem0, True)
    for q in range(6):
        wait_stage(q)
        lax.fori_loop(0, 8,
                      functools.partial(step2, buf=buf0, jbase=q * 8, p=q), 0)
        fire_stage(q)
    wait_stage(6)
    lax.fori_loop(0, 5,
                  functools.partial(step2, buf=buf0, jbase=48, p=6), 0)
    tail_cps = []
    for w in range(5):
        sl6 = pl.ds(6 * 128 + w * 16, 16)
        sl = pl.ds(w * 16, 16)
        tail_cps.append(pltpu.async_copy(
            d_src.at[sl6], out_el.at[ix[6][0][sl]], semsc[6]))
        tail_cps.append(pltpu.async_copy(
            d_dst.at[sl6], out_el.at[ix[6][1][sl]], semsc[6]))
        tail_cps.append(pltpu.async_copy(
            d_rel.at[sl6], out_el.at[ix[6][2][sl]], semsc[6]))
    wait_stage(7)
    for q in range(6):
        wait_stage(q)
    for cp in tail_cps:
        cp.wait()


@functools.cache
def _place_call():
    return pl.kernel(
        _place_body,
        out_type=(jax.ShapeDtypeStruct((E_TOT * 3,), jnp.int32),
                  jax.ShapeDtypeStruct((E_TOT,), jnp.int32),
                  jax.ShapeDtypeStruct((NGRAPH,), jnp.int32)),
        mesh=_mesh(),
        compiler_params=_SC_PARAMS,
        scratch_types=[
            pltpu.VMEM((N_NODES,), jnp.int32),
            pltpu.VMEM((16, NRB, 3), jnp.int32),
            pltpu.VMEM((16, NRB, 3), jnp.int32),
            pltpu.VMEM((16, NGRAPH), jnp.int32),
            pltpu.VMEM((NGRAPH,), jnp.int32),
            pltpu.VMEM((16, NGRAPH), jnp.int32),
            pltpu.VMEM((16, NGRAPH), jnp.int32),
            pltpu.VMEM((NGRAPH,), jnp.int32),
            pltpu.VMEM((NGRAPH,), jnp.int32),
            pltpu.VMEM((1024,), jnp.int32),
            pltpu.VMEM((1024,), jnp.int32),
        ] + [pltpu.VMEM((128,), jnp.int32) for _ in range(3 * NPAR)]
          + [pltpu.VMEM((NPAR * 128,), jnp.int32) for _ in range(3)]
          + [pltpu.SemaphoreType.DMA for _ in range(5 + NPAR)],
    )


def _mm_body(x_ref, w_ref, h_ref):
    h_ref[...] = jnp.maximum(
        jnp.dot(x_ref[...], w_ref[...], preferred_element_type=jnp.float32),
        0.0)


_mm_call = pl.pallas_call(
    _mm_body,
    grid=(50,),
    in_specs=[pl.BlockSpec((1000, DIM), lambda i: (i, 0)),
              pl.BlockSpec((DIM, DIM), lambda i: (0, 0))],
    out_specs=[pl.BlockSpec((1000, DIM), lambda i: (i, 0))],
    out_shape=[jax.ShapeDtypeStruct((N_NODES, DIM), jnp.float32)],
)


def _ones_body(ew_ref):
    ew_ref[...] = jnp.ones_like(ew_ref)


_ones_call = pl.pallas_call(
    _ones_body,
    grid=(1,),
    out_specs=[pl.BlockSpec((E_TOT // 128, 128), lambda i: (0, 0))],
    out_shape=[jax.ShapeDtypeStruct((E_TOT // 128, 128), jnp.float32)],
)


def kernel(x, W, node2graph, edge_list1, edge_list2):
    n2g = node2graph.astype(jnp.int32)
    el1 = edge_list1.astype(jnp.int32)
    el2 = edge_list2.astype(jnp.int32)
    hist = _hist_call()(el1, el2, n2g)
    elflat, offsets, num_edges = _place_call()(el1, el2, n2g, hist)
    (h,) = _mm_call(x, W)
    (ew2,) = _ones_call()
    edge_weight = ew2.reshape(-1)
    out_el = elflat.reshape(E_TOT, 3)
    num_relation = jnp.array(8, jnp.int32)
    return (h, out_el, edge_weight, num_edges, offsets, num_relation)


# speedup vs baseline: 4.4625x; 1.9140x over previous
import functools

import jax
import jax.numpy as jnp
from jax import lax
from jax.experimental import pallas as pl
from jax.experimental.pallas import tpu as pltpu
from jax.experimental.pallas import tpu_sc as plsc

N_NODES = 50000
NGRAPH = 128
DIM = 256
E_TOT = 1600000

NTILE = 32
CHUNK = E_TOT // NTILE
LCHUNK = CHUNK // 16
JB = 128
NBLK_FULL = LCHUNK // JB
TAIL = LCHUNK - NBLK_FULL * JB
D_AL = [(5 * l) % 8 for l in range(16)]
NRB = ((7 + JB + 7) // 8) * 8
TAIL_R = [((D_AL[l] + TAIL + 7) // 8) * 8 for l in range(16)]

_SC_PARAMS = pltpu.CompilerParams(needs_layout_passes=False,
                                  use_tc_tiling_on_sc=False)


@functools.cache
def _mesh():
    return plsc.VectorSubcoreMesh(core_axis_name="c", subcore_axis_name="s",
                                  num_cores=2, num_subcores=16)


def _lane_vecs():
    iota = lax.iota(jnp.int32, 16)
    dvec = (iota * 5) & 7
    return iota, dvec


def _fire_block(el1, el2, is1, base_r, j0, tail, buf, sem, cols):
    def fire(el):
        for l in range(16):
            r0 = base_r + (l * LCHUNK - D_AL[l]) + j0
            r0 = pl.multiple_of(r0, 8)
            nr = TAIL_R[l] if tail else NRB
            for cc in cols:
                pltpu.async_copy(el.at[cc, pl.ds(r0, nr)],
                                 buf.at[l, cc, pl.ds(0, nr)], sem)

    @pl.when(is1)
    def _():
        fire(el1)

    @pl.when(jnp.logical_not(is1))
    def _():
        fire(el2)


def _wait_block(el1, buf, sem, tail, cols):
    for l in range(16):
        nr = TAIL_R[l] if tail else NRB
        for cc in cols:
            pltpu.make_async_copy(el1.at[cc, pl.ds(0, nr)],
                                  buf.at[l, cc, pl.ds(0, nr)], sem).wait()


def _hist_body(el1, el2, n2g_hbm, hist_hbm, n2g_v, buf0, buf1, ctr,
               semn, sem0, sem1):
    c = lax.axis_index("c")
    s = lax.axis_index("s")
    t = c * 16 + s
    is1 = t < 16
    tloc = jnp.where(is1, t, t - 16)
    base_r = tloc * CHUNK

    cpn = pltpu.async_copy(n2g_hbm, n2g_v, semn)
    zz = jnp.zeros((16,), jnp.int32)
    for l in range(16):
        for bg in range(8):
            ctr[l, pl.ds(bg * 16, 16)] = zz

    iota, dvec = _lane_vecs()
    ones_i = jnp.full((16,), 1, jnp.int32)
    zeros16 = jnp.zeros((16,), jnp.int32)
    bufs = [buf0, buf1]
    sems = [sem0, sem1]
    _fire_block(el1, el2, is1, base_r, 0, False, buf0, sem0, (0,))
    _fire_block(el1, el2, is1, base_r, JB, False, buf1, sem1, (0,))
    cpn.wait()

    def hstep(j, carry, buf):
        src = plsc.load_gather(buf, [iota, zeros16, dvec + j])
        g = plsc.load_gather(n2g_v, [src])
        plsc.addupdate_scatter(ctr, [iota, g], ones_i)
        return carry

    def blk_body(i, carry):
        for b in range(2):
            kk = 2 * i + b
            _wait_block(el1, bufs[b], sems[b], False, (0,))
            lax.fori_loop(0, JB, functools.partial(hstep, buf=bufs[b]), 0)

            @pl.when(kk + 2 < NBLK_FULL)
            def _(kk=kk, b=b):
                _fire_block(el1, el2, is1, base_r, (kk + 2) * JB, False,
                            bufs[b], sems[b], (0,))

            @pl.when(kk == NBLK_FULL - 2)
            def _(b=b):
                _fire_block(el1, el2, is1, base_r, NBLK_FULL * JB, True,
                            bufs[b], sems[b], (0,))
        return carry

    lax.fori_loop(0, NBLK_FULL // 2, blk_body, 0)
    _wait_block(el1, buf0, sem0, True, (0,))
    lax.fori_loop(0, TAIL, functools.partial(hstep, buf=buf0), 0)
    pltpu.sync_copy(ctr, hist_hbm.at[t])


@functools.cache
def _hist_call():
    return pl.kernel(
        _hist_body,
        out_type=jax.ShapeDtypeStruct((NTILE, 16, NGRAPH), jnp.int32),
        mesh=_mesh(),
        compiler_params=_SC_PARAMS,
        scratch_types=[
            pltpu.VMEM((N_NODES,), jnp.int32),
            pltpu.VMEM((16, 3, NRB), jnp.int32),
            pltpu.VMEM((16, 3, NRB), jnp.int32),
            pltpu.VMEM((16, NGRAPH), jnp.int32),
            pltpu.SemaphoreType.DMA,
            pltpu.SemaphoreType.DMA,
            pltpu.SemaphoreType.DMA,
        ],
    )


NPAR = 8


def _place_body(el1, el2, n2g_hbm, hist_hbm,
                out_el, out_off, out_ne, *rest):
    it = iter(rest)
    n2g_v, buf0, buf1, ctr, starts_v, histp0, histp1, ne_stage, gb_v = (
        next(it) for _ in range(9))
    vfill = [next(it), next(it)]
    ix = [[next(it) for _ in range(3)] for _ in range(NPAR)]
    d_src, d_dst, d_rel = (next(it) for _ in range(3))
    semn, sem0, sem1, semh0, semh1 = (next(it) for _ in range(5))
    semsc = [next(it) for _ in range(NPAR)]
    c = lax.axis_index("c")
    s = lax.axis_index("s")
    t = c * 16 + s
    is1 = t < 16
    tloc = jnp.where(is1, t, t - 16)
    base_r = tloc * CHUNK
    iota, dvec = _lane_vecs()
    zeros16 = jnp.zeros((16,), jnp.int32)
    ones16 = jnp.full((16,), 1, jnp.int32)
    twos16 = jnp.full((16,), 2, jnp.int32)

    cpn = pltpu.async_copy(n2g_hbm, n2g_v, semn)

    histp = [histp0, histp1]
    semh = [semh0, semh1]
    pltpu.async_copy(hist_hbm.at[0], histp0, semh0)
    pltpu.async_copy(hist_hbm.at[1], histp1, semh1)

    def scan_body(i, acc):
        for b in range(2):
            tp = 2 * i + b
            pltpu.make_async_copy(hist_hbm.at[0], histp[b], semh[b]).wait()
            for l in range(16):
                @pl.when(tp == t)
                def _(l=l, acc=acc):
                    for bg in range(8):
                        ctr[l, pl.ds(bg * 16, 16)] = acc[bg]
                row = [histp[b][l, pl.ds(bg * 16, 16)] for bg in range(8)]
                acc = tuple(acc[bg] + row[bg] for bg in range(8))

            @pl.when(tp + 2 < NTILE)
            def _(b=b, tp=tp):
                pltpu.async_copy(hist_hbm.at[tp + 2], histp[b], semh[b])
        return acc

    acc = lax.fori_loop(0, NTILE // 2, scan_body,
                        tuple(zeros16 for _ in range(8)))

    for bg in range(8):
        ne_stage[pl.ds(bg * 16, 16)] = acc[bg]

    @pl.when(t == 0)
    def _():
        pltpu.sync_copy(ne_stage, out_ne)

    carry = jnp.int32(0)
    for bg in range(8):
        inc = plsc.cumsum(acc[bg])
        excl = inc - acc[bg] + carry
        carry = carry + jnp.sum(acc[bg])
        gb_v[pl.ds(bg * 16, 16)] = excl
        for l in range(16):
            ctr[l, pl.ds(bg * 16, 16)] = ctr[l, pl.ds(bg * 16, 16)] + excl

    cpn.wait()
    for bg in range(8):
        bvals = iota + bg * 16

        def bs_body(_, lohi, bvals=bvals):
            lo, hi = lohi
            mid = (lo + hi) >> 1
            v = plsc.load_gather(n2g_v, [mid])
            pred = v < bvals
            return (jnp.where(pred, mid + 1, lo), jnp.where(pred, hi, mid))

        lo, hi = lax.fori_loop(0, 17, bs_body,
                               (zeros16, jnp.full((16,), N_NODES, jnp.int32)))
        starts_v[pl.ds(bg * 16, 16)] = lo

    def vscal(ref, b):
        return jnp.max(plsc.load_gather(ref, [jnp.broadcast_to(b, (16,))]))

    for k in range(4):
        bn = tloc * 4 + k + jnp.where(is1, 0, 64)
        vb = vfill[k & 1]
        sem = semsc[k & 1]
        lo = vscal(gb_v, bn)
        n = vscal(ne_stage, bn)
        vv = plsc.load_gather(starts_v, [jnp.broadcast_to(bn, (16,))])

        def fillbuf(i, carry, vb=vb, vv=vv):
            plsc.store_scatter(vb, [i * 16 + iota], vv)
            return carry

        lax.fori_loop(0, 64, fillbuf, 0)

        @pl.when(n > 0)
        def _(bn=bn, vb=vb, sem=sem, lo=lo, n=n):
            hi = lo + n
            idxf = jnp.minimum(lo + iota, hi - 1)
            cpf = pltpu.async_copy(vb.at[pl.ds(0, 16)], out_off.at[idxf], sem)
            idxb = jnp.maximum(hi - 16 + iota, lo)
            cpb = pltpu.async_copy(vb.at[pl.ds(0, 16)], out_off.at[idxb], sem)
            lo_a = (lo + 15) & ~15
            hi_a = hi & ~15
            nch = jnp.maximum(hi_a - lo_a, 0)
            nbig = nch >> 10
            n256 = (nch & 1023) >> 8
            n16 = (nch & 255) >> 4

            def fbig(q, carry, vb=vb, sem=sem, lo_a=lo_a):
                d0 = pl.multiple_of(lo_a + q * 1024, 16)
                pltpu.async_copy(vb, out_off.at[pl.ds(d0, 1024)], sem)
                return carry

            def f256(q, carry, vb=vb, sem=sem, base=lo_a + (nbig << 10)):
                d0 = pl.multiple_of(base + q * 256, 16)
                pltpu.async_copy(vb.at[pl.ds(0, 256)],
                                 out_off.at[pl.ds(d0, 256)], sem)
                return carry

            def f16(q, carry, vb=vb, sem=sem,
                    base=lo_a + (nbig << 10) + (n256 << 8)):
                d0 = pl.multiple_of(base + q * 16, 16)
                pltpu.async_copy(vb.at[pl.ds(0, 16)],
                                 out_off.at[pl.ds(d0, 16)], sem)
                return carry

            lax.fori_loop(0, nbig, fbig, 0)
            lax.fori_loop(0, n256, f256, 0)
            lax.fori_loop(0, n16, f16, 0)

            def wbig(q, carry, vb=vb, sem=sem):
                pltpu.make_async_copy(vb, out_off.at[pl.ds(0, 1024)],
                                      sem).wait()
                return carry

            def w256(q, carry, vb=vb, sem=sem):
                pltpu.make_async_copy(vb.at[pl.ds(0, 256)],
                                      out_off.at[pl.ds(0, 256)], sem).wait()
                return carry

            def w16(q, carry, vb=vb, sem=sem):
                pltpu.make_async_copy(vb.at[pl.ds(0, 16)],
                                      out_off.at[pl.ds(0, 16)], sem).wait()
                return carry

            lax.fori_loop(0, nbig, wbig, 0)
            lax.fori_loop(0, n256, w256, 0)
            lax.fori_loop(0, n16, w16, 0)
            cpf.wait()
            cpb.wait()

    roff = jnp.where(is1, jnp.int32(0), jnp.int32(4))
    bufs = [buf0, buf1]
    sems = [sem0, sem1]

    def step2(j2, carry, buf, jbase, p):
        jj = jbase + j2
        rowv = dvec + jj
        src = plsc.load_gather(buf, [iota, zeros16, rowv])
        dst = plsc.load_gather(buf, [iota, ones16, rowv])
        rel = plsc.load_gather(buf, [iota, twos16, rowv]) + roff
        g = plsc.load_gather(n2g_v, [src])
        pos = plsc.load_gather(ctr, [iota, g])
        plsc.store_scatter(ctr, [iota, g], pos + 1)
        col = j2 * 16 + iota
        p3 = pos * 3
        plsc.store_scatter(ix[p][0], [col], p3)
        plsc.store_scatter(ix[p][1], [col], p3 + 1)
        plsc.store_scatter(ix[p][2], [col], p3 + 2)
        dcol = p * 128 + col
        plsc.store_scatter(d_src, [dcol], src)
        plsc.store_scatter(d_dst, [dcol], dst)
        plsc.store_scatter(d_rel, [dcol], rel)
        return carry

    def fire_stage(p):
        sl = pl.ds(p * 128, 128)
        return [pltpu.async_copy(d_src.at[sl], out_el.at[ix[p][0]], semsc[p]),
                pltpu.async_copy(d_dst.at[sl], out_el.at[ix[p][1]], semsc[p]),
                pltpu.async_copy(d_rel.at[sl], out_el.at[ix[p][2]], semsc[p])]

    def wait_stage(p):
        sl = pl.ds(p * 128, 128)
        pltpu.make_async_copy(d_src.at[sl], out_el.at[ix[p][0]],
                              semsc[p]).wait()
        pltpu.make_async_copy(d_dst.at[sl], out_el.at[ix[p][1]],
                              semsc[p]).wait()
        pltpu.make_async_copy(d_rel.at[sl], out_el.at[ix[p][2]],
                              semsc[p]).wait()

    _fire_block(el1, el2, is1, base_r, 0, False, buf0, sem0, (0, 1, 2))
    _fire_block(el1, el2, is1, base_r, JB, False, buf1, sem1, (0, 1, 2))

    NSG = JB // 8 // NPAR

    def blk_body(i, carry):
        for b in range(2):
            kk = 2 * i + b
            _wait_block(el1, bufs[b], sems[b], False, (0, 1, 2))

            def stage_grp(sg, carry2, buf=bufs[b], kk=kk):
                for p in range(NPAR):
                    gidx = kk * (JB // 8) + sg * NPAR + p

                    @pl.when(gidx >= NPAR)
                    def _(p=p):
                        wait_stage(p)

                    lax.fori_loop(0, 8,
                                  functools.partial(step2, buf=buf,
                                                    jbase=(sg * NPAR + p) * 8,
                                                    p=p),
                                  0)
                    fire_stage(p)
                return carry2

            lax.fori_loop(0, NSG, stage_grp, 0)

            @pl.when(kk + 2 < NBLK_FULL)
            def _(kk=kk, b=b):
                _fire_block(el1, el2, is1, base_r, (kk + 2) * JB, False,
                            bufs[b], sems[b], (0, 1, 2))

            @pl.when(kk == NBLK_FULL - 2)
            def _(b=b):
                _fire_block(el1, el2, is1, base_r, NBLK_FULL * JB, True,
                            bufs[b], sems[b], (0, 1, 2))
        return carry

    lax.fori_loop(0, NBLK_FULL // 2, blk_body, 0)

    _wait_block(el1, buf0, sem0, True, (0, 1, 2))
    for q in range(6):
        wait_stage(q)
        lax.fori_loop(0, 8,
                      functools.partial(step2, buf=buf0, jbase=q * 8, p=q), 0)
        fire_stage(q)
    wait_stage(6)
    lax.fori_loop(0, 5,
                  functools.partial(step2, buf=buf0, jbase=48, p=6), 0)
    tail_cps = []
    for w in range(5):
        sl6 = pl.ds(6 * 128 + w * 16, 16)
        sl = pl.ds(w * 16, 16)
        tail_cps.append(pltpu.async_copy(
            d_src.at[sl6], out_el.at[ix[6][0][sl]], semsc[6]))
        tail_cps.append(pltpu.async_copy(
            d_dst.at[sl6], out_el.at[ix[6][1][sl]], semsc[6]))
        tail_cps.append(pltpu.async_copy(
            d_rel.at[sl6], out_el.at[ix[6][2][sl]], semsc[6]))
    wait_stage(7)
    for q in range(6):
        wait_stage(q)
    for cp in tail_cps:
        cp.wait()


@functools.cache
def _place_call():
    return pl.kernel(
        _place_body,
        out_type=(jax.ShapeDtypeStruct((E_TOT * 3,), jnp.int32),
                  jax.ShapeDtypeStruct((E_TOT,), jnp.int32),
                  jax.ShapeDtypeStruct((NGRAPH,), jnp.int32)),
        mesh=_mesh(),
        compiler_params=_SC_PARAMS,
        scratch_types=[
            pltpu.VMEM((N_NODES,), jnp.int32),
            pltpu.VMEM((16, 3, NRB), jnp.int32),
            pltpu.VMEM((16, 3, NRB), jnp.int32),
            pltpu.VMEM((16, NGRAPH), jnp.int32),
            pltpu.VMEM((NGRAPH,), jnp.int32),
            pltpu.VMEM((16, NGRAPH), jnp.int32),
            pltpu.VMEM((16, NGRAPH), jnp.int32),
            pltpu.VMEM((NGRAPH,), jnp.int32),
            pltpu.VMEM((NGRAPH,), jnp.int32),
            pltpu.VMEM((1024,), jnp.int32),
            pltpu.VMEM((1024,), jnp.int32),
        ] + [pltpu.VMEM((128,), jnp.int32) for _ in range(3 * NPAR)]
          + [pltpu.VMEM((NPAR * 128,), jnp.int32) for _ in range(3)]
          + [pltpu.SemaphoreType.DMA for _ in range(5 + NPAR)],
    )


def _mm_body(x_ref, w_ref, h_ref):
    h_ref[...] = jnp.maximum(
        jnp.dot(x_ref[...], w_ref[...], preferred_element_type=jnp.float32),
        0.0)


_mm_call = pl.pallas_call(
    _mm_body,
    grid=(50,),
    in_specs=[pl.BlockSpec((1000, DIM), lambda i: (i, 0)),
              pl.BlockSpec((DIM, DIM), lambda i: (0, 0))],
    out_specs=[pl.BlockSpec((1000, DIM), lambda i: (i, 0))],
    out_shape=[jax.ShapeDtypeStruct((N_NODES, DIM), jnp.float32)],
)


def _ones_body(ew_ref):
    ew_ref[...] = jnp.ones_like(ew_ref)


_ones_call = pl.pallas_call(
    _ones_body,
    grid=(1,),
    out_specs=[pl.BlockSpec((E_TOT // 128, 128), lambda i: (0, 0))],
    out_shape=[jax.ShapeDtypeStruct((E_TOT // 128, 128), jnp.float32)],
)


def kernel(x, W, node2graph, edge_list1, edge_list2):
    n2g = node2graph.astype(jnp.int32)
    el1 = edge_list1.astype(jnp.int32).T
    el2 = edge_list2.astype(jnp.int32).T
    hist = _hist_call()(el1, el2, n2g)
    elflat, offsets, num_edges = _place_call()(el1, el2, n2g, hist)
    (h,) = _mm_call(x, W)
    (ew2,) = _ones_call()
    edge_weight = ew2.reshape(-1)
    out_el = elflat.reshape(E_TOT, 3)
    num_relation = jnp.array(8, jnp.int32)
    return (h, out_el, edge_weight, num_edges, offsets, num_relation)


# speedup vs baseline: 6.0126x; 1.3474x over previous
import functools

import jax
import jax.numpy as jnp
from jax import lax
from jax.experimental import pallas as pl
from jax.experimental.pallas import tpu as pltpu
from jax.experimental.pallas import tpu_sc as plsc

N_NODES = 50000
NGRAPH = 128
DIM = 256
E_TOT = 1600000

NTILE = 32
CHUNK = E_TOT // NTILE
LCHUNK = CHUNK // 16
JB = 128
NBLK_FULL = LCHUNK // JB
TAIL = LCHUNK - NBLK_FULL * JB
D_AL = [(5 * l) % 8 for l in range(16)]
NRB = ((7 + JB + 7) // 8) * 8
TAIL_R = [((D_AL[l] + TAIL + 7) // 8) * 8 for l in range(16)]

_SC_PARAMS = pltpu.CompilerParams(needs_layout_passes=False,
                                  use_tc_tiling_on_sc=False)


@functools.cache
def _mesh():
    return plsc.VectorSubcoreMesh(core_axis_name="c", subcore_axis_name="s",
                                  num_cores=2, num_subcores=16)


def _lane_vecs():
    iota = lax.iota(jnp.int32, 16)
    dvec = (iota * 5) & 7
    return iota, dvec


def _fire_block(el1, el2, is1, base_r, j0, tail, buf, sem, cols):
    def fire(el):
        for l in range(16):
            r0 = base_r + (l * LCHUNK - D_AL[l]) + j0
            r0 = pl.multiple_of(r0, 8)
            nr = TAIL_R[l] if tail else NRB
            for cc in cols:
                pltpu.async_copy(el.at[cc, pl.ds(r0, nr)],
                                 buf.at[l, cc, pl.ds(0, nr)], sem)

    @pl.when(is1)
    def _():
        fire(el1)

    @pl.when(jnp.logical_not(is1))
    def _():
        fire(el2)


def _wait_block(el1, buf, sem, tail, cols):
    for l in range(16):
        nr = TAIL_R[l] if tail else NRB
        for cc in cols:
            pltpu.make_async_copy(el1.at[cc, pl.ds(0, nr)],
                                  buf.at[l, cc, pl.ds(0, nr)], sem).wait()


def _hist_body(el1, el2, n2g_hbm, hist_hbm, n2g_v, buf0, buf1, ctr,
               semn, sem0, sem1):
    c = lax.axis_index("c")
    s = lax.axis_index("s")
    t = c * 16 + s
    is1 = t < 16
    tloc = jnp.where(is1, t, t - 16)
    base_r = tloc * CHUNK

    cpn = pltpu.async_copy(n2g_hbm, n2g_v, semn)
    zz = jnp.zeros((16,), jnp.int32)
    for l in range(16):
        for bg in range(8):
            ctr[l, pl.ds(bg * 16, 16)] = zz

    iota, dvec = _lane_vecs()
    ones_i = jnp.full((16,), 1, jnp.int32)
    zeros16 = jnp.zeros((16,), jnp.int32)
    bufs = [buf0, buf1]
    sems = [sem0, sem1]
    _fire_block(el1, el2, is1, base_r, 0, False, buf0, sem0, (0,))
    _fire_block(el1, el2, is1, base_r, JB, False, buf1, sem1, (0,))
    cpn.wait()

    def hstep(j, carry, buf):
        src = plsc.load_gather(buf, [iota, zeros16, dvec + j])
        g = plsc.load_gather(n2g_v, [src])
        plsc.addupdate_scatter(ctr, [iota, g], ones_i)
        return carry

    def blk_body(i, carry):
        for b in range(2):
            kk = 2 * i + b
            _wait_block(el1, bufs[b], sems[b], False, (0,))
            lax.fori_loop(0, JB, functools.partial(hstep, buf=bufs[b]), 0)

            @pl.when(kk + 2 < NBLK_FULL)
            def _(kk=kk, b=b):
                _fire_block(el1, el2, is1, base_r, (kk + 2) * JB, False,
                            bufs[b], sems[b], (0,))

            @pl.when(kk == NBLK_FULL - 2)
            def _(b=b):
                _fire_block(el1, el2, is1, base_r, NBLK_FULL * JB, True,
                            bufs[b], sems[b], (0,))
        return carry

    lax.fori_loop(0, NBLK_FULL // 2, blk_body, 0)
    _wait_block(el1, buf0, sem0, True, (0,))
    lax.fori_loop(0, TAIL, functools.partial(hstep, buf=buf0), 0)
    pltpu.sync_copy(ctr, hist_hbm.at[t])


@functools.cache
def _hist_call():
    return pl.kernel(
        _hist_body,
        out_type=jax.ShapeDtypeStruct((NTILE, 16, NGRAPH), jnp.int32),
        mesh=_mesh(),
        compiler_params=_SC_PARAMS,
        scratch_types=[
            pltpu.VMEM((N_NODES,), jnp.int32),
            pltpu.VMEM((16, 3, NRB), jnp.int32),
            pltpu.VMEM((16, 3, NRB), jnp.int32),
            pltpu.VMEM((16, NGRAPH), jnp.int32),
            pltpu.SemaphoreType.DMA,
            pltpu.SemaphoreType.DMA,
            pltpu.SemaphoreType.DMA,
        ],
    )


NPAR = 8


def _place_body(el1, el2, n2g_hbm, hist_hbm,
                out_el, out_off, out_ne, *rest):
    it = iter(rest)
    n2g_v, buf0, buf1, ctr, starts_v, histp0, histp1, ne_stage, gb_v = (
        next(it) for _ in range(9))
    vfill = [next(it), next(it)]
    ix = [[next(it) for _ in range(2)] for _ in range(NPAR)]
    d_src, d_dst = (next(it) for _ in range(2))
    semn, sem0, sem1, semh0, semh1 = (next(it) for _ in range(5))
    semsc = [next(it) for _ in range(NPAR)]
    c = lax.axis_index("c")
    s = lax.axis_index("s")
    t = c * 16 + s
    is1 = t < 16
    tloc = jnp.where(is1, t, t - 16)
    base_r = tloc * CHUNK
    iota, dvec = _lane_vecs()
    zeros16 = jnp.zeros((16,), jnp.int32)
    ones16 = jnp.full((16,), 1, jnp.int32)
    twos16 = jnp.full((16,), 2, jnp.int32)

    cpn = pltpu.async_copy(n2g_hbm, n2g_v, semn)

    histp = [histp0, histp1]
    semh = [semh0, semh1]
    pltpu.async_copy(hist_hbm.at[0], histp0, semh0)
    pltpu.async_copy(hist_hbm.at[1], histp1, semh1)

    def scan_body(i, acc):
        for b in range(2):
            tp = 2 * i + b
            pltpu.make_async_copy(hist_hbm.at[0], histp[b], semh[b]).wait()
            for l in range(16):
                @pl.when(tp == t)
                def _(l=l, acc=acc):
                    for bg in range(8):
                        ctr[l, pl.ds(bg * 16, 16)] = acc[bg]
                row = [histp[b][l, pl.ds(bg * 16, 16)] for bg in range(8)]
                acc = tuple(acc[bg] + row[bg] for bg in range(8))

            @pl.when(tp + 2 < NTILE)
            def _(b=b, tp=tp):
                pltpu.async_copy(hist_hbm.at[tp + 2], histp[b], semh[b])
        return acc

    acc = lax.fori_loop(0, NTILE // 2, scan_body,
                        tuple(zeros16 for _ in range(8)))

    for bg in range(8):
        ne_stage[pl.ds(bg * 16, 16)] = acc[bg]

    @pl.when(t == 0)
    def _():
        pltpu.sync_copy(ne_stage, out_ne)

    carry = jnp.int32(0)
    for bg in range(8):
        inc = plsc.cumsum(acc[bg])
        excl = inc - acc[bg] + carry
        carry = carry + jnp.sum(acc[bg])
        gb_v[pl.ds(bg * 16, 16)] = excl
        for l in range(16):
            ctr[l, pl.ds(bg * 16, 16)] = ctr[l, pl.ds(bg * 16, 16)] + excl

    cpn.wait()
    for bg in range(8):
        bvals = iota + bg * 16

        def bs_body(_, lohi, bvals=bvals):
            lo, hi = lohi
            mid = (lo + hi) >> 1
            v = plsc.load_gather(n2g_v, [mid])
            pred = v < bvals
            return (jnp.where(pred, mid + 1, lo), jnp.where(pred, hi, mid))

        lo, hi = lax.fori_loop(0, 17, bs_body,
                               (zeros16, jnp.full((16,), N_NODES, jnp.int32)))
        starts_v[pl.ds(bg * 16, 16)] = lo

    def vscal(ref, b):
        return jnp.max(plsc.load_gather(ref, [jnp.broadcast_to(b, (16,))]))

    for k in range(4):
        bn = tloc * 4 + k + jnp.where(is1, 0, 64)
        vb = vfill[k & 1]
        sem = semsc[k & 1]
        lo = vscal(gb_v, bn)
        n = vscal(ne_stage, bn)
        vv = plsc.load_gather(starts_v, [jnp.broadcast_to(bn, (16,))])

        def fillbuf(i, carry, vb=vb, vv=vv):
            plsc.store_scatter(vb, [i * 16 + iota], vv)
            return carry

        lax.fori_loop(0, 64, fillbuf, 0)

        @pl.when(n > 0)
        def _(bn=bn, vb=vb, sem=sem, lo=lo, n=n):
            hi = lo + n
            idxf = jnp.minimum(lo + iota, hi - 1)
            cpf = pltpu.async_copy(vb.at[pl.ds(0, 16)], out_off.at[idxf], sem)
            idxb = jnp.maximum(hi - 16 + iota, lo)
            cpb = pltpu.async_copy(vb.at[pl.ds(0, 16)], out_off.at[idxb], sem)
            lo_a = (lo + 15) & ~15
            hi_a = hi & ~15
            nch = jnp.maximum(hi_a - lo_a, 0)
            nbig = nch >> 10
            n256 = (nch & 1023) >> 8
            n16 = (nch & 255) >> 4

            def fbig(q, carry, vb=vb, sem=sem, lo_a=lo_a):
                d0 = pl.multiple_of(lo_a + q * 1024, 16)
                pltpu.async_copy(vb, out_off.at[pl.ds(d0, 1024)], sem)
                return carry

            def f256(q, carry, vb=vb, sem=sem, base=lo_a + (nbig << 10)):
                d0 = pl.multiple_of(base + q * 256, 16)
                pltpu.async_copy(vb.at[pl.ds(0, 256)],
                                 out_off.at[pl.ds(d0, 256)], sem)
                return carry

            def f16(q, carry, vb=vb, sem=sem,
                    base=lo_a + (nbig << 10) + (n256 << 8)):
                d0 = pl.multiple_of(base + q * 16, 16)
                pltpu.async_copy(vb.at[pl.ds(0, 16)],
                                 out_off.at[pl.ds(d0, 16)], sem)
                return carry

            lax.fori_loop(0, nbig, fbig, 0)
            lax.fori_loop(0, n256, f256, 0)
            lax.fori_loop(0, n16, f16, 0)

            def wbig(q, carry, vb=vb, sem=sem):
                pltpu.make_async_copy(vb, out_off.at[pl.ds(0, 1024)],
                                      sem).wait()
                return carry

            def w256(q, carry, vb=vb, sem=sem):
                pltpu.make_async_copy(vb.at[pl.ds(0, 256)],
                                      out_off.at[pl.ds(0, 256)], sem).wait()
                return carry

            def w16(q, carry, vb=vb, sem=sem):
                pltpu.make_async_copy(vb.at[pl.ds(0, 16)],
                                      out_off.at[pl.ds(0, 16)], sem).wait()
                return carry

            lax.fori_loop(0, nbig, wbig, 0)
            lax.fori_loop(0, n256, w256, 0)
            lax.fori_loop(0, n16, w16, 0)
            cpf.wait()
            cpb.wait()

    roff = jnp.where(is1, jnp.int32(0), jnp.int32(4))
    bufs = [buf0, buf1]
    sems = [sem0, sem1]

    def step2(j2, carry, buf, jbase, p):
        jj = jbase + j2
        rowv = dvec + jj
        src = plsc.load_gather(buf, [iota, zeros16, rowv])
        dst = plsc.load_gather(buf, [iota, ones16, rowv])
        rel = plsc.load_gather(buf, [iota, twos16, rowv]) + roff
        g = plsc.load_gather(n2g_v, [src])
        pos = plsc.load_gather(ctr, [iota, g])
        plsc.store_scatter(ctr, [iota, g], pos + 1)
        col = j2 * 16 + iota
        p2 = pos * 2
        plsc.store_scatter(ix[p][0], [col], p2)
        plsc.store_scatter(ix[p][1], [col], p2 + 1)
        dcol = p * 128 + col
        plsc.store_scatter(d_src, [dcol], (src << 16) | dst)
        plsc.store_scatter(d_dst, [dcol], rel)
        return carry

    def fire_stage(p):
        sl = pl.ds(p * 128, 128)
        return [pltpu.async_copy(d_src.at[sl], out_el.at[ix[p][0]], semsc[p]),
                pltpu.async_copy(d_dst.at[sl], out_el.at[ix[p][1]], semsc[p])]

    def wait_stage(p):
        sl = pl.ds(p * 128, 128)
        pltpu.make_async_copy(d_src.at[sl], out_el.at[ix[p][0]],
                              semsc[p]).wait()
        pltpu.make_async_copy(d_dst.at[sl], out_el.at[ix[p][1]],
                              semsc[p]).wait()

    _fire_block(el1, el2, is1, base_r, 0, False, buf0, sem0, (0, 1, 2))
    _fire_block(el1, el2, is1, base_r, JB, False, buf1, sem1, (0, 1, 2))

    NSG = JB // 8 // NPAR

    def blk_body(i, carry):
        for b in range(2):
            kk = 2 * i + b
            _wait_block(el1, bufs[b], sems[b], False, (0, 1, 2))

            def stage_grp(sg, carry2, buf=bufs[b], kk=kk):
                for p in range(NPAR):
                    gidx = kk * (JB // 8) + sg * NPAR + p

                    @pl.when(gidx >= NPAR)
                    def _(p=p):
                        wait_stage(p)

                    lax.fori_loop(0, 8,
                                  functools.partial(step2, buf=buf,
                                                    jbase=(sg * NPAR + p) * 8,
                                                    p=p),
                                  0)
                    fire_stage(p)
                return carry2

            lax.fori_loop(0, NSG, stage_grp, 0)

            @pl.when(kk + 2 < NBLK_FULL)
            def _(kk=kk, b=b):
                _fire_block(el1, el2, is1, base_r, (kk + 2) * JB, False,
                            bufs[b], sems[b], (0, 1, 2))

            @pl.when(kk == NBLK_FULL - 2)
            def _(b=b):
                _fire_block(el1, el2, is1, base_r, NBLK_FULL * JB, True,
                            bufs[b], sems[b], (0, 1, 2))
        return carry

    lax.fori_loop(0, NBLK_FULL // 2, blk_body, 0)

    _wait_block(el1, buf0, sem0, True, (0, 1, 2))
    for q in range(6):
        wait_stage(q)
        lax.fori_loop(0, 8,
                      functools.partial(step2, buf=buf0, jbase=q * 8, p=q), 0)
        fire_stage(q)
    wait_stage(6)
    lax.fori_loop(0, 5,
                  functools.partial(step2, buf=buf0, jbase=48, p=6), 0)
    tail_cps = []
    for w in range(5):
        sl6 = pl.ds(6 * 128 + w * 16, 16)
        sl = pl.ds(w * 16, 16)
        tail_cps.append(pltpu.async_copy(
            d_src.at[sl6], out_el.at[ix[6][0][sl]], semsc[6]))
        tail_cps.append(pltpu.async_copy(
            d_dst.at[sl6], out_el.at[ix[6][1][sl]], semsc[6]))
    wait_stage(7)
    for q in range(6):
        wait_stage(q)
    for cp in tail_cps:
        cp.wait()


@functools.cache
def _place_call():
    return pl.kernel(
        _place_body,
        out_type=(jax.ShapeDtypeStruct((E_TOT * 2,), jnp.int32),
                  jax.ShapeDtypeStruct((E_TOT,), jnp.int32),
                  jax.ShapeDtypeStruct((NGRAPH,), jnp.int32)),
        mesh=_mesh(),
        compiler_params=_SC_PARAMS,
        scratch_types=[
            pltpu.VMEM((N_NODES,), jnp.int32),
            pltpu.VMEM((16, 3, NRB), jnp.int32),
            pltpu.VMEM((16, 3, NRB), jnp.int32),
            pltpu.VMEM((16, NGRAPH), jnp.int32),
            pltpu.VMEM((NGRAPH,), jnp.int32),
            pltpu.VMEM((16, NGRAPH), jnp.int32),
            pltpu.VMEM((16, NGRAPH), jnp.int32),
            pltpu.VMEM((NGRAPH,), jnp.int32),
            pltpu.VMEM((NGRAPH,), jnp.int32),
            pltpu.VMEM((1024,), jnp.int32),
            pltpu.VMEM((1024,), jnp.int32),
        ] + [pltpu.VMEM((128,), jnp.int32) for _ in range(2 * NPAR)]
          + [pltpu.VMEM((NPAR * 128,), jnp.int32) for _ in range(2)]
          + [pltpu.SemaphoreType.DMA for _ in range(5 + NPAR)],
    )


UB = 4096
NBU = CHUNK * 2 // UB
REMW = CHUNK * 2 - NBU * UB


def _unpack_body(pk_hbm, elf_hbm, bin0, bin1, st0, st1, semi, semo):
    c = lax.axis_index("c")
    s = lax.axis_index("s")
    t = c * 16 + s
    iota = lax.iota(jnp.int32, 16)
    in0 = t * (CHUNK * 2)
    out0 = t * (CHUNK * 3)
    bins = [bin0, bin1]
    sts = [st0, st1]

    def fire_in(kk, b, n=UB):
        pltpu.async_copy(pk_hbm.at[pl.ds(in0 + kk * UB, n)],
                         bins[b].at[pl.ds(0, n)], semi)

    def wait_in(b, n=UB):
        pltpu.make_async_copy(pk_hbm.at[pl.ds(0, n)],
                              bins[b].at[pl.ds(0, n)], semi).wait()

    def wait_out(b, n=UB * 3 // 2):
        pltpu.make_async_copy(sts[b].at[pl.ds(0, n)],
                              elf_hbm.at[pl.ds(0, n)], semo).wait()

    def ustep(e, carry, bin_b, st_b):
        ev = e * 16 + iota
        w1 = plsc.load_gather(bin_b, [ev * 2])
        w2 = plsc.load_gather(bin_b, [ev * 2 + 1])
        e3 = ev * 3
        plsc.store_scatter(st_b, [e3], lax.shift_right_logical(w1, 16))
        plsc.store_scatter(st_b, [e3 + 1], w1 & 0xFFFF)
        plsc.store_scatter(st_b, [e3 + 2], w2)
        return carry

    fire_in(0, 0)
    fire_in(1, 1)

    def blk(i, carry):
        for b in range(2):
            kk = 2 * i + b
            wait_in(b)

            @pl.when(kk >= 2)
            def _(b=b):
                wait_out(b)

            lax.fori_loop(0, UB // 32, functools.partial(
                ustep, bin_b=bins[b], st_b=sts[b]), 0)
            pltpu.async_copy(
                sts[b].at[pl.ds(0, UB * 3 // 2)],
                elf_hbm.at[pl.ds(out0 + kk * (UB * 3 // 2), UB * 3 // 2)],
                semo)

            @pl.when(kk + 2 < NBU)
            def _(kk=kk, b=b):
                fire_in(kk + 2, b)
        return carry

    lax.fori_loop(0, NBU // 2, blk, 0)
    fire_in(NBU, 0, REMW)
    wait_in(0, REMW)
    wait_out(0)
    lax.fori_loop(0, REMW // 32, functools.partial(
        ustep, bin_b=bins[0], st_b=sts[0]), 0)
    cp = pltpu.async_copy(
        sts[0].at[pl.ds(0, REMW * 3 // 2)],
        elf_hbm.at[pl.ds(out0 + NBU * (UB * 3 // 2), REMW * 3 // 2)], semo)
    wait_out(1)
    cp.wait()


@functools.cache
def _unpack_call():
    return pl.kernel(
        _unpack_body,
        out_type=jax.ShapeDtypeStruct((E_TOT * 3,), jnp.int32),
        mesh=_mesh(),
        compiler_params=_SC_PARAMS,
        scratch_types=[
            pltpu.VMEM((UB,), jnp.int32),
            pltpu.VMEM((UB,), jnp.int32),
            pltpu.VMEM((UB * 3 // 2,), jnp.int32),
            pltpu.VMEM((UB * 3 // 2,), jnp.int32),
            pltpu.SemaphoreType.DMA,
            pltpu.SemaphoreType.DMA,
        ],
    )


def _mm_body(x_ref, w_ref, h_ref):
    h_ref[...] = jnp.maximum(
        jnp.dot(x_ref[...], w_ref[...], preferred_element_type=jnp.float32),
        0.0)


_mm_call = pl.pallas_call(
    _mm_body,
    grid=(50,),
    in_specs=[pl.BlockSpec((1000, DIM), lambda i: (i, 0)),
              pl.BlockSpec((DIM, DIM), lambda i: (0, 0))],
    out_specs=[pl.BlockSpec((1000, DIM), lambda i: (i, 0))],
    out_shape=[jax.ShapeDtypeStruct((N_NODES, DIM), jnp.float32)],
)


def _ones_body(ew_ref):
    ew_ref[...] = jnp.ones_like(ew_ref)


_ones_call = pl.pallas_call(
    _ones_body,
    grid=(1,),
    out_specs=[pl.BlockSpec((E_TOT // 128, 128), lambda i: (0, 0))],
    out_shape=[jax.ShapeDtypeStruct((E_TOT // 128, 128), jnp.float32)],
)


def kernel(x, W, node2graph, edge_list1, edge_list2):
    n2g = node2graph.astype(jnp.int32)
    el1 = edge_list1.astype(jnp.int32).T
    el2 = edge_list2.astype(jnp.int32).T
    hist = _hist_call()(el1, el2, n2g)
    pk, offsets, num_edges = _place_call()(el1, el2, n2g, hist)
    elflat = _unpack_call()(pk)
    (h,) = _mm_call(x, W)
    (ew2,) = _ones_call()
    edge_weight = ew2.reshape(-1)
    out_el = elflat.reshape(E_TOT, 3)
    num_relation = jnp.array(8, jnp.int32)
    return (h, out_el, edge_weight, num_edges, offsets, num_relation)
